# Initial kernel scaffold; baseline (speedup 1.0000x reference)
import jax
import jax.numpy as jnp
from jax.experimental import pallas as pl


def kernel(item_ids, cat_ids, amount, timestamp, seq_lens, item_table, cat_table):
    raise NotImplementedError("write your pallas kernel here")


# speedup vs baseline: 1.9438x; 1.9438x over previous
import functools

import jax
import jax.numpy as jnp
from jax import lax
from jax.experimental import pallas as pl
from jax.experimental.pallas import tpu as pltpu
from jax.experimental.pallas import tpu_sc as plsc

_D = 32
_DO = 2 * _D + 2
_GRP = 4
_NC = 2
_NS = 16
_NW = _NC * _NS
_L = 16


@functools.lru_cache(maxsize=None)
def _build(N, L, V_item, V_cat):
    chunk = L
    cpad = -(-chunk // _L) * _L
    rows_per_w = N // _NW
    n_chunks = rows_per_w // chunk
    n_groups = cpad // _L

    def body(item_idx, cat_idx, amt, ts, item_tab, cat_tab, out,
             ii_v, ci_v, si_v, sc_v, amt_v, ts_v, gi_buf, gc_buf, buf,
             sem_i, sem_c):
        wid = lax.axis_index("s") * _NC + lax.axis_index("c")
        lanes = lax.iota(jnp.int32, _L)
        tail_n = chunk - (n_groups - 1) * _L
        col_a = jnp.full((_L,), 2 * _D, jnp.int32)
        col_d = jnp.full((_L,), 2 * _D + 1, jnp.int32)

        def prep_idx(raw_v, sub_v, vmax):
            for g in range(n_groups):
                s = pl.ds(g * _L, _L)
                v = jnp.clip(raw_v[s], 0, vmax - 1)
                if g == n_groups - 1:
                    v = jnp.where(lanes < tail_n, v, 0)
                sub_v[s] = (v & (_GRP - 1)) << 5
                raw_v[s] = v >> 2

        def chunk_body(c, _):
            base = wid * rows_per_w + c * chunk
            pltpu.sync_copy(item_idx.at[pl.ds(base, chunk)],
                            ii_v.at[pl.ds(0, chunk)])
            pltpu.sync_copy(cat_idx.at[pl.ds(base, chunk)],
                            ci_v.at[pl.ds(0, chunk)])
            prep_idx(ii_v, si_v, V_item)
            prep_idx(ci_v, sc_v, V_cat)
            cp_i = pltpu.async_copy(item_tab.at[ii_v], gi_buf, sem_i)
            cp_c = pltpu.async_copy(cat_tab.at[ci_v], gc_buf, sem_c)
            pltpu.sync_copy(amt.at[pl.ds(base, chunk)],
                            amt_v.at[pl.ds(0, chunk)])
            pltpu.sync_copy(ts.at[pl.ds(base, chunk)],
                            ts_v.at[pl.ds(0, chunk)])
            for g in range(n_groups):
                o = g * _L + lanes
                msk = (lanes < tail_n) if g == n_groups - 1 else None
                a = ts_v[pl.ds(g * _L, _L)]
                pidx = o - 1
                if g == 0:
                    pidx = jnp.maximum(pidx, 0)
                d = a - plsc.load_gather(ts_v, [pidx])
                if g == 0:
                    d = jnp.where(lanes == 0, jnp.float32(0), d)
                av = amt_v[pl.ds(g * _L, _L)]
                plsc.store_scatter(buf, [o, col_a], av, mask=msk)
                plsc.store_scatter(buf, [o, col_d], d, mask=msk)
            cp_i.wait()
            cp_c.wait()
            for g in range(n_groups):
                rows = g * _L + lanes
                msk = (lanes < tail_n) if g == n_groups - 1 else None
                si = si_v[pl.ds(g * _L, _L)]
                sc = sc_v[pl.ds(g * _L, _L)]
                for col in range(_D):
                    vi = plsc.load_gather(gi_buf, [rows, si + col])
                    plsc.store_scatter(buf, [rows, jnp.full((_L,), col,
                                                            jnp.int32)],
                                       vi, mask=msk)
                    vc = plsc.load_gather(gc_buf, [rows, sc + col])
                    plsc.store_scatter(buf, [rows, jnp.full((_L,), _D + col,
                                                            jnp.int32)],
                                       vc, mask=msk)
            pltpu.sync_copy(buf, out.at[pl.ds(base, chunk)])
            return 0

        lax.fori_loop(0, n_chunks, chunk_body, 0)

    return pl.kernel(
        body,
        out_type=jax.ShapeDtypeStruct((N, _DO), jnp.float32),
        mesh=plsc.VectorSubcoreMesh(core_axis_name="c", subcore_axis_name="s",
                                    num_cores=_NC, num_subcores=_NS),
        scratch_types=[
            pltpu.VMEM((cpad,), jnp.int32),
            pltpu.VMEM((cpad,), jnp.int32),
            pltpu.VMEM((cpad,), jnp.int32),
            pltpu.VMEM((cpad,), jnp.int32),
            pltpu.VMEM((cpad,), jnp.float32),
            pltpu.VMEM((cpad,), jnp.float32),
            pltpu.VMEM((cpad, _GRP * _D), jnp.float32),
            pltpu.VMEM((cpad, _GRP * _D), jnp.float32),
            pltpu.VMEM((chunk, _DO), jnp.float32),
            pltpu.SemaphoreType.DMA,
            pltpu.SemaphoreType.DMA,
        ],
        compiler_params=pltpu.CompilerParams(needs_layout_passes=False),
    )


def kernel(item_ids, cat_ids, amount, timestamp, seq_lens, item_table,
           cat_table):
    del seq_lens
    B, L = item_ids.shape
    N = B * L
    ii = item_ids.reshape(N).astype(jnp.int32)
    ci = cat_ids.reshape(N).astype(jnp.int32)
    am = amount.reshape(N).astype(jnp.float32)
    ts = timestamp.reshape(N).astype(jnp.float32)
    tab_i = item_table.reshape(item_table.shape[0] // _GRP, _GRP * _D)
    tab_c = cat_table.reshape(cat_table.shape[0] // _GRP, _GRP * _D)
    fn = _build(N, L, item_table.shape[0], cat_table.shape[0])
    out = fn(ii, ci, am, ts, tab_i, tab_c)
    return out.reshape(B, L, _DO)


# speedup vs baseline: 2.1945x; 1.1290x over previous
import functools

import jax
import jax.numpy as jnp
from jax import lax
from jax.experimental import pallas as pl
from jax.experimental.pallas import tpu as pltpu
from jax.experimental.pallas import tpu_sc as plsc

_D = 32
_DO = 2 * _D + 2
_GRP = 4
_NC = 2
_NS = 16
_NW = _NC * _NS
_L = 16


@functools.lru_cache(maxsize=None)
def _build(N, L, V_item, V_cat):
    chunk = L
    cpad = -(-chunk // _L) * _L
    rows_per_w = N // _NW
    n_chunks = rows_per_w // chunk
    n_groups = cpad // _L

    def body(item_idx, cat_idx, amt, ts, item_tab, cat_tab, out,
             ii_v, ci_v, si_v, sc_v, amt_v, ts_v, gi_buf, gc_buf, buf,
             sem_i, sem_c):
        wid = lax.axis_index("s") * _NC + lax.axis_index("c")
        lanes = lax.iota(jnp.int32, _L)
        tail_n = chunk - (n_groups - 1) * _L
        col_a = jnp.full((_L,), 2 * _D, jnp.int32)
        col_d = jnp.full((_L,), 2 * _D + 1, jnp.int32)

        def prep_idx(raw_v, sub_v, vmax):
            for g in range(n_groups):
                s = pl.ds(g * _L, _L)
                v = jnp.clip(raw_v[s], 0, vmax - 1)
                if g == n_groups - 1:
                    v = jnp.where(lanes < tail_n, v, 0)
                sub_v[s] = (v & (_GRP - 1)) << 5
                raw_v[s] = v >> 2

        def chunk_body(c, _):
            base = wid * rows_per_w + c * chunk
            pltpu.sync_copy(item_idx.at[pl.ds(base, chunk)],
                            ii_v.at[pl.ds(0, chunk)])
            pltpu.sync_copy(cat_idx.at[pl.ds(base, chunk)],
                            ci_v.at[pl.ds(0, chunk)])
            prep_idx(ii_v, si_v, V_item)
            prep_idx(ci_v, sc_v, V_cat)
            cp_i = pltpu.async_copy(item_tab.at[ii_v], gi_buf, sem_i)
            cp_c = pltpu.async_copy(cat_tab.at[ci_v], gc_buf, sem_c)
            pltpu.sync_copy(amt.at[pl.ds(base, chunk)],
                            amt_v.at[pl.ds(0, chunk)])
            pltpu.sync_copy(ts.at[pl.ds(base, chunk)],
                            ts_v.at[pl.ds(0, chunk)])
            for g in range(n_groups):
                o = g * _L + lanes
                msk = (lanes < tail_n) if g == n_groups - 1 else None
                a = ts_v[pl.ds(g * _L, _L)]
                pidx = o - 1
                if g == 0:
                    pidx = jnp.maximum(pidx, 0)
                d = a - plsc.load_gather(ts_v, [pidx])
                if g == 0:
                    d = jnp.where(lanes == 0, jnp.float32(0), d)
                av = amt_v[pl.ds(g * _L, _L)]
                plsc.store_scatter(buf, [o, col_a], av, mask=msk)
                plsc.store_scatter(buf, [o, col_d], d, mask=msk)
            cp_i.wait()
            cp_c.wait()

            def row_body(r, _):
                oi = si_v[pl.ds(r, _L)][0]
                oc = sc_v[pl.ds(r, _L)][0]
                for h in range(2):
                    buf[r, pl.ds(h * _L, _L)] = gi_buf[r, pl.ds(oi + h * _L,
                                                                _L)]
                    buf[r, pl.ds(_D + h * _L, _L)] = gc_buf[r,
                                                            pl.ds(oc + h * _L,
                                                                  _L)]
                return 0

            lax.fori_loop(0, chunk, row_body, 0, unroll=2)
            pltpu.sync_copy(buf, out.at[pl.ds(base, chunk)])
            return 0

        lax.fori_loop(0, n_chunks, chunk_body, 0)

    return pl.kernel(
        body,
        out_type=jax.ShapeDtypeStruct((N, _DO), jnp.float32),
        mesh=plsc.VectorSubcoreMesh(core_axis_name="c", subcore_axis_name="s",
                                    num_cores=_NC, num_subcores=_NS),
        scratch_types=[
            pltpu.VMEM((cpad,), jnp.int32),
            pltpu.VMEM((cpad,), jnp.int32),
            pltpu.VMEM((cpad + _L,), jnp.int32),
            pltpu.VMEM((cpad + _L,), jnp.int32),
            pltpu.VMEM((cpad,), jnp.float32),
            pltpu.VMEM((cpad,), jnp.float32),
            pltpu.VMEM((cpad, _GRP * _D), jnp.float32),
            pltpu.VMEM((cpad, _GRP * _D), jnp.float32),
            pltpu.VMEM((chunk, _DO), jnp.float32),
            pltpu.SemaphoreType.DMA,
            pltpu.SemaphoreType.DMA,
        ],
        compiler_params=pltpu.CompilerParams(needs_layout_passes=False),
    )


def kernel(item_ids, cat_ids, amount, timestamp, seq_lens, item_table,
           cat_table):
    del seq_lens
    B, L = item_ids.shape
    N = B * L
    ii = item_ids.reshape(N).astype(jnp.int32)
    ci = cat_ids.reshape(N).astype(jnp.int32)
    am = amount.reshape(N).astype(jnp.float32)
    ts = timestamp.reshape(N).astype(jnp.float32)
    tab_i = item_table.reshape(item_table.shape[0] // _GRP, _GRP * _D)
    tab_c = cat_table.reshape(cat_table.shape[0] // _GRP, _GRP * _D)
    fn = _build(N, L, item_table.shape[0], cat_table.shape[0])
    out = fn(ii, ci, am, ts, tab_i, tab_c)
    return out.reshape(B, L, _DO)


# speedup vs baseline: 4.5250x; 2.0620x over previous
import functools

import jax
import jax.numpy as jnp
from jax import lax
from jax.experimental import pallas as pl
from jax.experimental.pallas import tpu as pltpu
from jax.experimental.pallas import tpu_sc as plsc

_D = 32
_DO = 2 * _D + 2
_GRP = 4
_NC = 2
_NS = 16
_NW = _NC * _NS
_L = 16
_SC = 8
_H0 = 104


@functools.lru_cache(maxsize=None)
def _build(N, L, V_item, V_cat):
    chunk = L
    h1 = chunk - _H0
    srows = _SC * chunk
    rows_per_w = N // _NW
    n_supers = rows_per_w // srows
    n_groups = -(-chunk // _L)
    prep_groups = srows // _L

    def body(item_idx, cat_idx, amt, ts, item_tab, cat_tab, out,
             ii_s, ci_s, amt_s, ts_s, sisc_s,
             gi_a, gc_a, gi_b, gc_b, buf0, buf1,
             sem_ia, sem_ca, sem_ib, sem_cb, sem_o0, sem_o1):
        wid = lax.axis_index("s") * _NC + lax.axis_index("c")
        lanes = lax.iota(jnp.int32, _L)
        tail_n = chunk - (n_groups - 1) * _L
        col_a = jnp.full((_L,), 2 * _D, jnp.int32)
        col_d = jnp.full((_L,), 2 * _D + 1, jnp.int32)
        bufs = (buf0, buf1)
        sems_o = (sem_o0, sem_o1)
        gsets = ((gi_a, gc_a, sem_ia, sem_ca),
                 (gi_b, gc_b, sem_ib, sem_cb))

        def prep_body(g, _):
            s = pl.ds(g * _L, _L)
            vi = jnp.clip(ii_s[s], 0, V_item - 1)
            vc = jnp.clip(ci_s[s], 0, V_cat - 1)
            si = (vi & (_GRP - 1)) << 5
            sc = (vc & (_GRP - 1)) << 5
            sisc_s[s] = si | (sc << 8)
            ii_s[s] = vi >> 2
            ci_s[s] = vc >> 2
            return 0

        def issue_gather(t):
            k, parity = divmod(t, 2)
            off = k * chunk + (_H0 if parity else 0)
            n = h1 if parity else _H0
            gi, gc, s_i, s_c = gsets[parity]
            cp_i = pltpu.async_copy(
                item_tab.at[ii_s.at[pl.ds(off, n)]], gi, s_i)
            cp_c = pltpu.async_copy(
                cat_tab.at[ci_s.at[pl.ds(off, n)]], gc, s_c)
            return cp_i, cp_c

        def assemble(t, b):
            k, parity = divmod(t, 2)
            off = k * chunk + (_H0 if parity else 0)
            hb = _H0 if parity else 0
            n = h1 if parity else _H0
            gi, gc, _, _ = gsets[parity]

            def row_body(r, _):
                v = sisc_s[pl.ds(off + r, _L)][0]
                oi = v & 0xFF
                oc = v >> 8
                for h in range(2):
                    b[hb + r, pl.ds(h * _L, _L)] = gi[r, pl.ds(oi + h * _L,
                                                               _L)]
                    b[hb + r, pl.ds(_D + h * _L, _L)] = gc[r,
                                                           pl.ds(oc + h * _L,
                                                                 _L)]
                return 0

            lax.fori_loop(0, n, row_body, 0, unroll=2)

        def numeric(k, b):
            off = k * chunk
            for g in range(n_groups):
                o = g * _L + lanes
                msk = (lanes < tail_n) if g == n_groups - 1 else None
                a = ts_s[pl.ds(off + g * _L, _L)]
                pidx = off + o - 1
                if g == 0 and k == 0:
                    pidx = jnp.maximum(pidx, 0)
                d = a - plsc.load_gather(ts_s, [pidx])
                if g == 0:
                    d = jnp.where(lanes == 0, jnp.float32(0), d)
                av = amt_s[pl.ds(off + g * _L, _L)]
                plsc.store_scatter(b, [o, col_a], av, mask=msk)
                plsc.store_scatter(b, [o, col_d], d, mask=msk)

        def super_body(sup, _):
            sbase = wid * rows_per_w + sup * srows
            pltpu.sync_copy(item_idx.at[pl.ds(sbase, srows)],
                            ii_s.at[pl.ds(0, srows)])
            pltpu.sync_copy(cat_idx.at[pl.ds(sbase, srows)],
                            ci_s.at[pl.ds(0, srows)])
            pltpu.sync_copy(amt.at[pl.ds(sbase, srows)],
                            amt_s.at[pl.ds(0, srows)])
            pltpu.sync_copy(ts.at[pl.ds(sbase, srows)],
                            ts_s.at[pl.ds(0, srows)])
            lax.fori_loop(0, prep_groups, prep_body, 0)

            cps = issue_gather(0)
            out_cps = [None, None]
            for t in range(2 * _SC):
                k, parity = divmod(t, 2)
                cb = k % 2
                nxt = issue_gather(t + 1) if t + 1 < 2 * _SC else None
                cps[0].wait()
                cps[1].wait()
                b = bufs[cb]
                if parity == 0 and out_cps[cb] is not None:
                    out_cps[cb].wait()
                assemble(t, b)
                if parity == 1:
                    numeric(k, b)
                    out_cps[cb] = pltpu.async_copy(
                        b, out.at[pl.ds(sbase + k * chunk, chunk)],
                        sems_o[cb])
                cps = nxt
            out_cps[0].wait()
            out_cps[1].wait()
            return 0

        lax.fori_loop(0, n_supers, super_body, 0)

    return pl.kernel(
        body,
        out_type=jax.ShapeDtypeStruct((N, _DO), jnp.float32),
        mesh=plsc.VectorSubcoreMesh(core_axis_name="c", subcore_axis_name="s",
                                    num_cores=_NC, num_subcores=_NS),
        scratch_types=[
            pltpu.VMEM((_SC * L,), jnp.int32),
            pltpu.VMEM((_SC * L,), jnp.int32),
            pltpu.VMEM((_SC * L + _L,), jnp.float32),
            pltpu.VMEM((_SC * L + _L,), jnp.float32),
            pltpu.VMEM((_SC * L + _L,), jnp.int32),
            pltpu.VMEM((_H0, _GRP * _D), jnp.float32),
            pltpu.VMEM((_H0, _GRP * _D), jnp.float32),
            pltpu.VMEM((L - _H0, _GRP * _D), jnp.float32),
            pltpu.VMEM((L - _H0, _GRP * _D), jnp.float32),
            pltpu.VMEM((L, _DO), jnp.float32),
            pltpu.VMEM((L, _DO), jnp.float32),
            pltpu.SemaphoreType.DMA,
            pltpu.SemaphoreType.DMA,
            pltpu.SemaphoreType.DMA,
            pltpu.SemaphoreType.DMA,
            pltpu.SemaphoreType.DMA,
            pltpu.SemaphoreType.DMA,
        ],
        compiler_params=pltpu.CompilerParams(needs_layout_passes=False),
    )


def kernel(item_ids, cat_ids, amount, timestamp, seq_lens, item_table,
           cat_table):
    del seq_lens
    B, L = item_ids.shape
    N = B * L
    ii = item_ids.reshape(N).astype(jnp.int32)
    ci = cat_ids.reshape(N).astype(jnp.int32)
    am = amount.reshape(N).astype(jnp.float32)
    ts = timestamp.reshape(N).astype(jnp.float32)
    tab_i = item_table.reshape(item_table.shape[0] // _GRP, _GRP * _D)
    tab_c = cat_table.reshape(cat_table.shape[0] // _GRP, _GRP * _D)
    fn = _build(N, L, item_table.shape[0], cat_table.shape[0])
    out = fn(ii, ci, am, ts, tab_i, tab_c)
    return out.reshape(B, L, _DO)
